# Initial kernel scaffold; baseline (speedup 1.0000x reference)
import jax
import jax.numpy as jnp
from jax.experimental import pallas as pl


def kernel(byte_ids, segment_ids, emb_table, W, b):
    raise NotImplementedError("write your pallas kernel here")


# speedup vs baseline: 10.8197x; 10.8197x over previous
import functools

import jax
import jax.numpy as jnp
from jax import lax
from jax.experimental import pallas as pl
from jax.experimental.pallas import tpu as pltpu
from jax.experimental.pallas import tpu_sc as plsc

TOTAL_TOKENS = 32768
NUM_PATCHES = 2048
EMBED_DIM = 1024
PATCH_DIM = 512
VOCAB = 256

NC = 2
NS = 16
LANES = 16

TPW = TOTAL_TOKENS // (NC * NS)
HWORDS = NUM_PATCHES * VOCAB
HSLICE = HWORDS // NS
SCHUNK = 128


def _hist_body(seg_hbm, byte_hbm, zeros_hbm, out_hbm,
               seg_v, byte_v, idx_v, ones_v, hist_sh):
    cid = lax.axis_index("c")
    sid = lax.axis_index("s")
    base = (cid * NS + sid) * TPW

    pltpu.sync_copy(zeros_hbm.at[pl.ds(sid * HSLICE, HSLICE)],
                    hist_sh.at[pl.ds(sid * HSLICE, HSLICE)])

    pltpu.sync_copy(seg_hbm.at[pl.ds(base, TPW)], seg_v)
    pltpu.sync_copy(byte_hbm.at[pl.ds(base, TPW)], byte_v)

    for i in range(SCHUNK // LANES):
        ones_v[pl.ds(i * LANES, LANES)] = jnp.ones((LANES,), jnp.float32)

    for j in range(TPW // SCHUNK):
        for k in range(SCHUNK // LANES):
            off = j * SCHUNK + k * LANES
            seg16 = seg_v[pl.ds(off, LANES)]
            byt16 = byte_v[pl.ds(off, LANES)]
            idx_v[j, pl.ds(k * LANES, LANES)] = seg16 * VOCAB + byt16

    plsc.subcore_barrier()

    for j in range(TPW // SCHUNK):
        pltpu.sync_copy(ones_v, hist_sh.at[idx_v.at[j]], add=True)

    plsc.subcore_barrier()

    pltpu.sync_copy(hist_sh.at[pl.ds(sid * HSLICE, HSLICE)],
                    out_hbm.at[cid, sid])


_hist_kernel = functools.partial(
    pl.kernel,
    out_type=jax.ShapeDtypeStruct((NC, NS, HSLICE), jnp.float32),
    mesh=plsc.VectorSubcoreMesh(core_axis_name="c", subcore_axis_name="s",
                                num_cores=NC, num_subcores=NS),
    scratch_types=[
        pltpu.VMEM((TPW,), jnp.int32),
        pltpu.VMEM((TPW,), jnp.int32),
        pltpu.VMEM((TPW // SCHUNK, SCHUNK), jnp.int32),
        pltpu.VMEM((SCHUNK,), jnp.float32),
        pltpu.VMEM_SHARED((HWORDS,), jnp.float32),
    ],
)(_hist_body)


def _encode_body(h_ref, e_ref, w_ref, b_ref, o_ref):
    ew = jnp.dot(e_ref[...], w_ref[...],
                 preferred_element_type=jnp.float32,
                 precision=lax.Precision.HIGHEST)
    h = h_ref[0] + h_ref[1]
    counts = jnp.sum(h, axis=1, keepdims=True)
    acc = jnp.dot(h, ew,
                  preferred_element_type=jnp.float32,
                  precision=lax.Precision.HIGHEST)
    o_ref[...] = acc / jnp.maximum(counts, 1.0) + b_ref[...]


def kernel(byte_ids, segment_ids, emb_table, W, b):
    zeros = jnp.zeros((HWORDS,), jnp.float32)
    h_partial = _hist_kernel(segment_ids, byte_ids, zeros)
    h_partial = h_partial.reshape(NC, NUM_PATCHES, VOCAB)
    encoded = pl.pallas_call(
        _encode_body,
        out_shape=jax.ShapeDtypeStruct((NUM_PATCHES, PATCH_DIM), jnp.float32),
    )(h_partial, emb_table, W, b.reshape(1, PATCH_DIM))
    return encoded[None]


# speedup vs baseline: 12.0311x; 1.1120x over previous
import functools

import jax
import jax.numpy as jnp
from jax import lax
from jax.experimental import pallas as pl
from jax.experimental.pallas import tpu as pltpu
from jax.experimental.pallas import tpu_sc as plsc

TOTAL_TOKENS = 32768
NUM_PATCHES = 2048
EMBED_DIM = 1024
PATCH_DIM = 512
VOCAB = 256

NC = 2
NS = 16
LANES = 16

TPW = TOTAL_TOKENS // (NC * NS)
HWORDS = NUM_PATCHES * VOCAB
HSLICE = HWORDS // NS
SCHUNK = 128


def _hist_body(seg_hbm, byte_hbm, zeros_hbm, out_hbm,
               seg_v, byte_v, idx_v, ones_v, hist_sh, dma_sem):
    cid = lax.axis_index("c")
    sid = lax.axis_index("s")
    base = (cid * NS + sid) * TPW

    zdesc = pltpu.async_copy(zeros_hbm.at[pl.ds(sid * HSLICE, HSLICE)],
                             hist_sh.at[pl.ds(sid * HSLICE, HSLICE)],
                             dma_sem)
    pltpu.sync_copy(seg_hbm.at[pl.ds(base, TPW)], seg_v)
    pltpu.sync_copy(byte_hbm.at[pl.ds(base, TPW)], byte_v)

    for i in range(SCHUNK // LANES):
        ones_v[pl.ds(i * LANES, LANES)] = jnp.ones((LANES,), jnp.float32)

    for j in range(TPW // SCHUNK):
        for k in range(SCHUNK // LANES):
            off = j * SCHUNK + k * LANES
            seg16 = seg_v[pl.ds(off, LANES)]
            byt16 = byte_v[pl.ds(off, LANES)]
            idx_v[j, pl.ds(k * LANES, LANES)] = seg16 * VOCAB + byt16

    zdesc.wait()
    plsc.subcore_barrier()

    descs = [pltpu.async_copy(ones_v, hist_sh.at[idx_v.at[j]], dma_sem,
                              add=True)
             for j in range(TPW // SCHUNK)]
    for d in descs:
        d.wait()

    plsc.subcore_barrier()

    pltpu.sync_copy(hist_sh.at[pl.ds(sid * HSLICE, HSLICE)],
                    out_hbm.at[cid, sid])


_hist_kernel = functools.partial(
    pl.kernel,
    out_type=jax.ShapeDtypeStruct((NC, NS, HSLICE), jnp.float32),
    mesh=plsc.VectorSubcoreMesh(core_axis_name="c", subcore_axis_name="s",
                                num_cores=NC, num_subcores=NS),
    scratch_types=[
        pltpu.VMEM((TPW,), jnp.int32),
        pltpu.VMEM((TPW,), jnp.int32),
        pltpu.VMEM((TPW // SCHUNK, SCHUNK), jnp.int32),
        pltpu.VMEM((SCHUNK,), jnp.float32),
        pltpu.VMEM_SHARED((HWORDS,), jnp.float32),
        pltpu.SemaphoreType.DMA,
    ],
)(_hist_body)


def _encode_body(h_ref, e_ref, w_ref, b_ref, o_ref):
    ew = jnp.dot(e_ref[...], w_ref[...],
                 preferred_element_type=jnp.float32,
                 precision=lax.Precision.HIGHEST)
    h = h_ref[0] + h_ref[1]
    counts = jnp.sum(h, axis=1, keepdims=True)
    acc = jnp.dot(h, ew,
                  preferred_element_type=jnp.float32)
    o_ref[...] = acc / jnp.maximum(counts, 1.0) + b_ref[...]


def kernel(byte_ids, segment_ids, emb_table, W, b):
    zeros = jnp.zeros((HWORDS,), jnp.float32)
    h_partial = _hist_kernel(segment_ids, byte_ids, zeros)
    h_partial = h_partial.reshape(NC, NUM_PATCHES, VOCAB)
    encoded = pl.pallas_call(
        _encode_body,
        out_shape=jax.ShapeDtypeStruct((NUM_PATCHES, PATCH_DIM), jnp.float32),
    )(h_partial, emb_table, W, b.reshape(1, PATCH_DIM))
    return encoded[None]


# speedup vs baseline: 13.9152x; 1.1566x over previous
import functools

import jax
import jax.numpy as jnp
from jax import lax
from jax.experimental import pallas as pl
from jax.experimental.pallas import tpu as pltpu
from jax.experimental.pallas import tpu_sc as plsc

TOTAL_TOKENS = 32768
NUM_PATCHES = 2048
EMBED_DIM = 1024
PATCH_DIM = 512
VOCAB = 256

NC = 2
NS = 16
LANES = 16

TPW = TOTAL_TOKENS // (NC * NS)
HWORDS = NUM_PATCHES * VOCAB
HSLICE = HWORDS // NS
SCHUNK = 128


def _hist_body(seg_hbm, byte_hbm, zeros_hbm, out_hbm,
               seg_v, byte_v, idx_v, ones_v, hist_sh, dma_sem):
    cid = lax.axis_index("c")
    sid = lax.axis_index("s")
    base = (cid * NS + sid) * TPW

    zdesc = pltpu.async_copy(zeros_hbm.at[pl.ds(sid * HSLICE, HSLICE)],
                             hist_sh.at[pl.ds(sid * HSLICE, HSLICE)],
                             dma_sem)
    pltpu.sync_copy(seg_hbm.at[pl.ds(base, TPW)], seg_v)
    pltpu.sync_copy(byte_hbm.at[pl.ds(base, TPW)], byte_v)

    for i in range(SCHUNK // LANES):
        ones_v[pl.ds(i * LANES, LANES)] = jnp.ones((LANES,), jnp.float32)

    for j in range(TPW // SCHUNK):
        for k in range(SCHUNK // LANES):
            off = j * SCHUNK + k * LANES
            seg16 = seg_v[pl.ds(off, LANES)]
            byt16 = byte_v[pl.ds(off, LANES)]
            idx_v[j, pl.ds(k * LANES, LANES)] = seg16 * VOCAB + byt16

    zdesc.wait()
    plsc.subcore_barrier()

    descs = [pltpu.async_copy(ones_v, hist_sh.at[idx_v.at[j]], dma_sem,
                              add=True)
             for j in range(TPW // SCHUNK)]
    for d in descs:
        d.wait()

    plsc.subcore_barrier()

    pltpu.sync_copy(hist_sh.at[pl.ds(sid * HSLICE, HSLICE)],
                    out_hbm.at[cid, sid])


_hist_kernel = functools.partial(
    pl.kernel,
    out_type=jax.ShapeDtypeStruct((NC, NS, HSLICE), jnp.float32),
    mesh=plsc.VectorSubcoreMesh(core_axis_name="c", subcore_axis_name="s",
                                num_cores=NC, num_subcores=NS),
    scratch_types=[
        pltpu.VMEM((TPW,), jnp.int32),
        pltpu.VMEM((TPW,), jnp.int32),
        pltpu.VMEM((TPW // SCHUNK, SCHUNK), jnp.int32),
        pltpu.VMEM((SCHUNK,), jnp.float32),
        pltpu.VMEM_SHARED((HWORDS,), jnp.float32),
        pltpu.SemaphoreType.DMA,
    ],
)(_hist_body)


def _encode_body(h_ref, e_ref, w_ref, b_ref, o_ref):
    ew = jnp.dot(e_ref[...], w_ref[...],
                 preferred_element_type=jnp.float32,
                 precision=lax.Precision.HIGHEST)
    hs = h_ref[0] + h_ref[1]
    h = hs.reshape(NUM_PATCHES, VOCAB)
    counts = jnp.sum(h, axis=1, keepdims=True)
    acc = jnp.dot(h, ew,
                  preferred_element_type=jnp.float32)
    o_ref[...] = acc / jnp.maximum(counts, 1.0) + b_ref[...]


def kernel(byte_ids, segment_ids, emb_table, W, b):
    zeros = jnp.zeros((HWORDS,), jnp.float32)
    h_partial = _hist_kernel(segment_ids, byte_ids, zeros)
    encoded = pl.pallas_call(
        _encode_body,
        out_shape=jax.ShapeDtypeStruct((NUM_PATCHES, PATCH_DIM), jnp.float32),
    )(h_partial, emb_table, W, b.reshape(1, PATCH_DIM))
    return encoded[None]


# speedup vs baseline: 15.4136x; 1.1077x over previous
import functools

import jax
import jax.numpy as jnp
from jax import lax
from jax.experimental import pallas as pl
from jax.experimental.pallas import tpu as pltpu
from jax.experimental.pallas import tpu_sc as plsc

TOTAL_TOKENS = 32768
NUM_PATCHES = 2048
EMBED_DIM = 1024
PATCH_DIM = 512
VOCAB = 256

NC = 2
NS = 16
LANES = 16

TPW = TOTAL_TOKENS // (NC * NS)
HWORDS = NUM_PATCHES * VOCAB
HSLICE = HWORDS // NS
SCHUNK = 128


def _hist_body(seg_hbm, byte_hbm, zeros_hbm, out_hbm,
               seg_v, byte_v, idx_v, ones_v, hist_sh, dma_sem):
    cid = lax.axis_index("c")
    sid = lax.axis_index("s")
    base = (cid * NS + sid) * TPW

    zdesc = pltpu.async_copy(zeros_hbm.at[pl.ds(sid * HSLICE, HSLICE)],
                             hist_sh.at[pl.ds(sid * HSLICE, HSLICE)],
                             dma_sem)
    pltpu.sync_copy(seg_hbm.at[pl.ds(base, TPW)], seg_v)
    pltpu.sync_copy(byte_hbm.at[pl.ds(base, TPW)], byte_v)

    for i in range(SCHUNK // LANES):
        ones_v[pl.ds(i * LANES, LANES)] = jnp.ones((LANES,), jnp.float32)

    for j in range(TPW // SCHUNK):
        for k in range(SCHUNK // LANES):
            off = j * SCHUNK + k * LANES
            seg16 = seg_v[pl.ds(off, LANES)]
            byt16 = byte_v[pl.ds(off, LANES)]
            idx_v[j, pl.ds(k * LANES, LANES)] = seg16 * VOCAB + byt16

    zdesc.wait()
    plsc.subcore_barrier()

    descs = [pltpu.async_copy(ones_v, hist_sh.at[idx_v.at[j]], dma_sem,
                              add=True)
             for j in range(TPW // SCHUNK)]
    for d in descs:
        d.wait()

    plsc.subcore_barrier()

    pltpu.sync_copy(hist_sh.at[pl.ds(sid * HSLICE, HSLICE)],
                    out_hbm.at[cid, sid])


_hist_kernel = functools.partial(
    pl.kernel,
    out_type=jax.ShapeDtypeStruct((NC, NS, HSLICE), jnp.float32),
    mesh=plsc.VectorSubcoreMesh(core_axis_name="c", subcore_axis_name="s",
                                num_cores=NC, num_subcores=NS),
    scratch_types=[
        pltpu.VMEM((TPW,), jnp.int32),
        pltpu.VMEM((TPW,), jnp.int32),
        pltpu.VMEM((TPW // SCHUNK, SCHUNK), jnp.int32),
        pltpu.VMEM((SCHUNK,), jnp.float32),
        pltpu.VMEM_SHARED((HWORDS,), jnp.float32),
        pltpu.SemaphoreType.DMA,
    ],
)(_hist_body)


def _ew_body(e_ref, w_ref, o_ref):
    o_ref[...] = jnp.dot(e_ref[...], w_ref[...],
                         preferred_element_type=jnp.float32,
                         precision=lax.Precision.HIGHEST)


PBLK = 1024
GRID = NUM_PATCHES // PBLK
ROWS_PER_BLK = PBLK // (VOCAB // 2)


def _encode_body(ew_ref, b_ref, h_ref, o_ref):
    hs = h_ref[0] + h_ref[1]
    h = hs.reshape(PBLK, VOCAB)
    counts = jnp.sum(h, axis=1, keepdims=True)
    acc = jnp.dot(h, ew_ref[...],
                  preferred_element_type=jnp.float32)
    o_ref[...] = acc / jnp.maximum(counts, 1.0) + b_ref[...]


_ZEROS_NP = __import__("numpy").zeros((HWORDS,), "float32")


def kernel(byte_ids, segment_ids, emb_table, W, b):
    zeros = jnp.asarray(_ZEROS_NP)
    ew = pl.pallas_call(
        _ew_body,
        out_shape=jax.ShapeDtypeStruct((VOCAB, PATCH_DIM), jnp.float32),
    )(emb_table, W)
    h_partial = _hist_kernel(segment_ids, byte_ids, zeros)
    encoded = pl.pallas_call(
        _encode_body,
        grid=(GRID,),
        in_specs=[
            pl.BlockSpec((VOCAB, PATCH_DIM), lambda i: (0, 0)),
            pl.BlockSpec((1, PATCH_DIM), lambda i: (0, 0)),
            pl.BlockSpec((NC, ROWS_PER_BLK, TOTAL_TOKENS), lambda i: (0, i, 0)),
        ],
        out_specs=pl.BlockSpec((PBLK, PATCH_DIM), lambda i: (i, 0)),
        out_shape=jax.ShapeDtypeStruct((NUM_PATCHES, PATCH_DIM), jnp.float32),
    )(ew, b.reshape(1, PATCH_DIM), h_partial)
    return encoded[None]


# speedup vs baseline: 16.2840x; 1.0565x over previous
import functools

import numpy as np

import jax
import jax.numpy as jnp
from jax import lax
from jax.experimental import pallas as pl
from jax.experimental.pallas import tpu as pltpu
from jax.experimental.pallas import tpu_sc as plsc

TOTAL_TOKENS = 32768
NUM_PATCHES = 2048
EMBED_DIM = 1024
PATCH_DIM = 512
VOCAB = 256

NC = 2
NS = 16
LANES = 16

TPW = TOTAL_TOKENS // (NC * NS)
HWORDS = NUM_PATCHES * (VOCAB // 2)
HSLICE = HWORDS // NS
SCHUNK = 128


def _hist_body(seg_hbm, byte_hbm, zeros_hbm, out_hbm,
               seg_v, byte_v, idx_v, val_v, hist_sh, dma_sem):
    cid = lax.axis_index("c")
    sid = lax.axis_index("s")
    base = (cid * NS + sid) * TPW

    zdesc = pltpu.async_copy(zeros_hbm,
                             hist_sh.at[pl.ds(sid * HSLICE, HSLICE)],
                             dma_sem)
    pltpu.sync_copy(seg_hbm.at[pl.ds(base, TPW)], seg_v)
    pltpu.sync_copy(byte_hbm.at[pl.ds(base, TPW)], byte_v)

    for j in range(TPW // SCHUNK):
        for k in range(SCHUNK // LANES):
            off = j * SCHUNK + k * LANES
            seg16 = seg_v[pl.ds(off, LANES)]
            byt16 = byte_v[pl.ds(off, LANES)]
            idx_v[j, pl.ds(k * LANES, LANES)] = (
                seg16 * (VOCAB // 2) + lax.shift_right_logical(byt16, 1))
            val_v[j, pl.ds(k * LANES, LANES)] = (
                1 + lax.shift_left(byt16 & 1, 16))

    zdesc.wait()
    plsc.subcore_barrier()

    descs = [pltpu.async_copy(val_v.at[j], hist_sh.at[idx_v.at[j]], dma_sem,
                              add=True)
             for j in range(TPW // SCHUNK)]
    for d in descs:
        d.wait()

    plsc.subcore_barrier()

    pltpu.sync_copy(hist_sh.at[pl.ds(sid * HSLICE, HSLICE)],
                    out_hbm.at[cid, sid])


_hist_kernel = functools.partial(
    pl.kernel,
    out_type=jax.ShapeDtypeStruct((NC, NS, HSLICE), jnp.int32),
    mesh=plsc.VectorSubcoreMesh(core_axis_name="c", subcore_axis_name="s",
                                num_cores=NC, num_subcores=NS),
    scratch_types=[
        pltpu.VMEM((TPW,), jnp.int32),
        pltpu.VMEM((TPW,), jnp.int32),
        pltpu.VMEM((TPW // SCHUNK, SCHUNK), jnp.int32),
        pltpu.VMEM((TPW // SCHUNK, SCHUNK), jnp.int32),
        pltpu.VMEM_SHARED((HWORDS,), jnp.int32),
        pltpu.SemaphoreType.DMA,
    ],
)(_hist_body)


def _ew_body(ee_ref, eo_ref, w_ref, oe_ref, oo_ref):
    oe_ref[...] = jnp.dot(ee_ref[...], w_ref[...],
                          preferred_element_type=jnp.float32,
                          precision=lax.Precision.HIGHEST)
    oo_ref[...] = jnp.dot(eo_ref[...], w_ref[...],
                          preferred_element_type=jnp.float32,
                          precision=lax.Precision.HIGHEST)


PBLK = 1024
GRID = NUM_PATCHES // PBLK
ROWS_PER_BLK = NS // GRID


def _encode_body(ewe_ref, ewo_ref, b_ref, h_ref, o_ref):
    hs = h_ref[0] + h_ref[1]
    p = hs.reshape(PBLK, VOCAB // 2)
    lo = (p & 0xFFFF).astype(jnp.float32)
    hi = lax.shift_right_logical(p, 16).astype(jnp.float32)
    counts = (jnp.sum(lo, axis=1, keepdims=True)
              + jnp.sum(hi, axis=1, keepdims=True))
    acc = (jnp.dot(lo, ewe_ref[...], preferred_element_type=jnp.float32)
           + jnp.dot(hi, ewo_ref[...], preferred_element_type=jnp.float32))
    o_ref[...] = acc / jnp.maximum(counts, 1.0) + b_ref[...]


_ZEROS_NP = np.zeros((HSLICE,), np.int32)


def kernel(byte_ids, segment_ids, emb_table, W, b):
    zeros = jnp.asarray(_ZEROS_NP)
    ew_even, ew_odd = pl.pallas_call(
        _ew_body,
        out_shape=(
            jax.ShapeDtypeStruct((VOCAB // 2, PATCH_DIM), jnp.float32),
            jax.ShapeDtypeStruct((VOCAB // 2, PATCH_DIM), jnp.float32),
        ),
    )(emb_table[0::2], emb_table[1::2], W)
    h_packed = _hist_kernel(segment_ids, byte_ids, zeros)
    encoded = pl.pallas_call(
        _encode_body,
        grid=(GRID,),
        in_specs=[
            pl.BlockSpec((VOCAB // 2, PATCH_DIM), lambda i: (0, 0)),
            pl.BlockSpec((VOCAB // 2, PATCH_DIM), lambda i: (0, 0)),
            pl.BlockSpec((1, PATCH_DIM), lambda i: (0, 0)),
            pl.BlockSpec((NC, ROWS_PER_BLK, HSLICE), lambda i: (0, i, 0)),
        ],
        out_specs=pl.BlockSpec((PBLK, PATCH_DIM), lambda i: (i, 0)),
        out_shape=jax.ShapeDtypeStruct((NUM_PATCHES, PATCH_DIM), jnp.float32),
    )(ew_even, ew_odd, b.reshape(1, PATCH_DIM), h_packed)
    return encoded[None]
